# baseline (speedup 1.0000x reference)
import functools

import jax
import jax.numpy as jnp
from jax import lax
from jax.experimental import pallas as pl
from jax.experimental.pallas import tpu as pltpu
from jax.experimental.pallas import tpu_sc as plsc

NUM_ENTITIES = 100000
NUM_RELATIONS = 64
NUM_TIMESTAMPS = 1024
EMBED_DIM = 64
BATCH = 16384

NC = 2
NS = 16
NW = NC * NS
B_PER_W = BATCH // NW
CHUNK = 128
NCHUNK = B_PER_W // CHUNK
NOPS = 2
NTASK = NOPS * NCHUNK
NBUF = 4
PADDED = 128
NTC = BATCH // CHUNK

_MESH = plsc.VectorSubcoreMesh(core_axis_name="c", subcore_axis_name="s")

_OUT_SHAPE = jax.ShapeDtypeStruct((8, NTC * 8, CHUNK), jnp.float32)

_SCRATCH = (
    [pltpu.VMEM((NTASK, CHUNK), jnp.int32)]
    + [pltpu.VMEM((CHUNK, PADDED), jnp.float32) for _ in range(NBUF)]
    + [pltpu.VMEM((8, 8, 129), jnp.float32) for _ in range(NBUF)]
    + [pltpu.SemaphoreType.DMA for _ in range(2 * NBUF + NOPS)]
)


@functools.partial(
    pl.kernel,
    out_type=(_OUT_SHAPE, _OUT_SHAPE),
    mesh=_MESH,
    scratch_types=_SCRATCH,
    compiler_params=pltpu.CompilerParams(
        use_tc_tiling_on_sc=False, needs_layout_passes=False),
)
def _pair_gather(h2, t2, tbl_a, tbl_b,
                 out_h, out_t, idx_v,
                   b0, b1, b2, b3,
                   t0, t1, t2b, t3,
                   g0, g1, g2, g3,
                   s0, s1, s2s, s3,
                   i0, i1):
    wid = lax.axis_index("s") * NC + lax.axis_index("c")
    rowbase = wid * NCHUNK

    bufs = [b0, b1, b2, b3]
    tbufs = [t0, t1, t2b, t3]
    gsems = [g0, g1, g2, g3]
    ssems = [s0, s1, s2s, s3]
    isems = [i0, i1]
    idx_srcs = [h2, t2]
    tbls = [tbl_a, tbl_b]
    outs = [out_h, out_t]

    icopies = []
    for op in range(NOPS):
        icopies.append(pltpu.async_copy(
            idx_srcs[op].at[pl.ds(rowbase, NCHUNK)],
            idx_v.at[pl.ds(op * NCHUNK, NCHUNK)], isems[op]))
    idx_ready = [False] * NOPS

    tasks = [
        (outs[op], op, op * NCHUNK + c, c)
        for op in range(NOPS)
        for c in range(NCHUNK)
    ]

    gcopies = [None] * NTASK
    scopies = [None] * NTASK

    def start_gather(p):
        _, op, irow, _ = tasks[p]
        if not idx_ready[op]:
            icopies[op].wait()
            idx_ready[op] = True
        gcopies[p] = pltpu.async_copy(
            tbls[op].at[idx_v.at[irow]], bufs[p % NBUF], gsems[p % NBUF])

    iota16 = lax.iota(jnp.int32, 16)
    dgroups = []
    for d0 in range(0, EMBED_DIM, 16):
        dvec = d0 + iota16
        dgroups.append((dvec, dvec // 8, dvec % 8))

    def transpose_chunk(buf, tbuf):
        def body(c, carry):
            cc = jnp.full((16,), c, jnp.int32)
            for dvec, trh, srh in dgroups:
                v = plsc.load_gather(buf, [cc, dvec])
                plsc.store_scatter(tbuf, [trh, srh, cc], v)
            return carry
        lax.fori_loop(0, CHUNK, body, jnp.int32(0), unroll=4)

    def start_store(p):
        out_p, _, _, c = tasks[p]
        gcopies[p].wait()
        transpose_chunk(bufs[p % NBUF], tbufs[p % NBUF])
        tcg = wid * NCHUNK + c
        scopies[p] = pltpu.async_copy(
            tbufs[p % NBUF].at[:, :, pl.ds(0, CHUNK)],
            out_p.at[:, pl.ds(tcg * 8, 8), :],
            ssems[p % NBUF])

    for step in range(NTASK):
        if step >= NBUF:
            scopies[step - NBUF].wait()
        start_gather(step)
        d = step - (NBUF - 1)
        if d >= 0:
            start_store(d)
    for d in range(NTASK - NBUF + 1, NTASK):
        start_store(d)
    for d in range(NTASK - NBUF, NTASK):
        scopies[d].wait()


_ENT_BLK = 8192
_ENT_GRID = -(-NUM_ENTITIES // _ENT_BLK)


def _ent_repack_body(tin, dep_a, dep_b, tout):
    del dep_a, dep_b
    x = tin[...]
    p = (lax.broadcasted_iota(jnp.int32, (EMBED_DIM, PADDED), 0)
         == lax.broadcasted_iota(jnp.int32, (EMBED_DIM, PADDED), 1)
         ).astype(jnp.float32)
    tout[...] = lax.dot_general(
        x, p, (((0,), (0,)), ((), ())),
        precision=lax.Precision.HIGHEST,
        preferred_element_type=jnp.float32)


_ent_repack = pl.pallas_call(
    _ent_repack_body,
    grid=(_ENT_GRID,),
    in_specs=[pl.BlockSpec((EMBED_DIM, _ENT_BLK), lambda i: (0, i)),
              pl.BlockSpec((NUM_RELATIONS, PADDED), lambda i: (0, 0)),
              pl.BlockSpec((NUM_TIMESTAMPS, PADDED), lambda i: (0, 0))],
    out_specs=pl.BlockSpec((_ENT_BLK, PADDED), lambda i: (i, 0)),
    out_shape=jax.ShapeDtypeStruct((NUM_ENTITIES, PADDED), jnp.float32),
)

def _untile(o):
    o4 = o.reshape(8, NTC, 8, CHUNK)
    return o4.transpose(1, 3, 0, 2).reshape(BATCH, EMBED_DIM)


def kernel(head, relation, tail, timestamp,
           entity_table, relation_table, timestamp_table):
    shp = (NTC, CHUNK)
    h2 = head.astype(jnp.int32).reshape(shp)
    t2 = tail.astype(jnp.int32).reshape(shp)
    r2 = relation.astype(jnp.int32).reshape(shp)
    s2 = timestamp.astype(jnp.int32).reshape(shp)

    pad = ((0, 0), (0, PADDED - EMBED_DIM))
    rel_p = jnp.pad(relation_table, pad)
    ts_p = jnp.pad(timestamp_table, pad)
    ent_lin = _ent_repack(entity_table.T, rel_p, ts_p)
    o_rel, o_ts = _pair_gather(r2, s2, rel_p, ts_p)
    o_head, o_tail = _pair_gather(h2, t2, ent_lin, ent_lin)

    return (_untile(o_head), _untile(o_rel), _untile(o_tail), _untile(o_ts))


# speedup vs baseline: 1.0007x; 1.0007x over previous
import functools

import jax
import jax.numpy as jnp
from jax import lax
from jax.experimental import pallas as pl
from jax.experimental.pallas import tpu as pltpu
from jax.experimental.pallas import tpu_sc as plsc

NUM_ENTITIES = 100000
NUM_RELATIONS = 64
NUM_TIMESTAMPS = 1024
EMBED_DIM = 64
BATCH = 16384

NC = 2
NS = 16
NW = NC * NS
B_PER_W = BATCH // NW
CHUNK = 128
NCHUNK = B_PER_W // CHUNK
NOPS = 2
NTASK = NOPS * NCHUNK
NBUF = 4
PADDED = 128
NTC = BATCH // CHUNK

_MESH = plsc.VectorSubcoreMesh(core_axis_name="c", subcore_axis_name="s")

_OUT_SHAPE = jax.ShapeDtypeStruct((8, NTC * 8, CHUNK), jnp.float32)

_SCRATCH = (
    [pltpu.VMEM((NTASK, CHUNK), jnp.int32)]
    + [pltpu.VMEM((CHUNK, PADDED), jnp.float32) for _ in range(NBUF)]
    + [pltpu.VMEM((8, 8, 129), jnp.float32) for _ in range(NBUF)]
    + [pltpu.SemaphoreType.DMA for _ in range(2 * NBUF + NOPS)]
)


@functools.partial(
    pl.kernel,
    out_type=(_OUT_SHAPE, _OUT_SHAPE),
    mesh=_MESH,
    scratch_types=_SCRATCH,
    compiler_params=pltpu.CompilerParams(
        use_tc_tiling_on_sc=False, needs_layout_passes=False),
)
def _pair_gather(h2, t2, tbl_a, tbl_b,
                 out_h, out_t, idx_v,
                   b0, b1, b2, b3,
                   t0, t1, t2b, t3,
                   g0, g1, g2, g3,
                   s0, s1, s2s, s3,
                   i0, i1):
    wid = lax.axis_index("s") * NC + lax.axis_index("c")
    rowbase = wid * NCHUNK

    bufs = [b0, b1, b2, b3]
    tbufs = [t0, t1, t2b, t3]
    gsems = [g0, g1, g2, g3]
    ssems = [s0, s1, s2s, s3]
    isems = [i0, i1]
    idx_srcs = [h2, t2]
    tbls = [tbl_a, tbl_b]
    outs = [out_h, out_t]

    icopies = []
    for op in range(NOPS):
        icopies.append(pltpu.async_copy(
            idx_srcs[op].at[pl.ds(rowbase, NCHUNK)],
            idx_v.at[pl.ds(op * NCHUNK, NCHUNK)], isems[op]))
    idx_ready = [False] * NOPS

    tasks = [
        (outs[op], op, op * NCHUNK + c, c)
        for op in range(NOPS)
        for c in range(NCHUNK)
    ]

    gcopies = [None] * NTASK
    scopies = [None] * NTASK

    def start_gather(p):
        _, op, irow, _ = tasks[p]
        if not idx_ready[op]:
            icopies[op].wait()
            idx_ready[op] = True
        gcopies[p] = pltpu.async_copy(
            tbls[op].at[idx_v.at[irow]], bufs[p % NBUF], gsems[p % NBUF])

    iota16 = lax.iota(jnp.int32, 16)
    dgroups = []
    for d0 in range(0, EMBED_DIM, 16):
        dvec = d0 + iota16
        dgroups.append((dvec, dvec // 8, dvec % 8))

    def transpose_chunk(buf, tbuf):
        def body(c, carry):
            cc = jnp.full((16,), c, jnp.int32)
            for dvec, trh, srh in dgroups:
                v = plsc.load_gather(buf, [cc, dvec])
                plsc.store_scatter(tbuf, [trh, srh, cc], v)
            return carry
        lax.fori_loop(0, CHUNK, body, jnp.int32(0), unroll=4)

    def start_store(p):
        out_p, _, _, c = tasks[p]
        gcopies[p].wait()
        transpose_chunk(bufs[p % NBUF], tbufs[p % NBUF])
        tcg = wid * NCHUNK + c
        scopies[p] = pltpu.async_copy(
            tbufs[p % NBUF].at[:, :, pl.ds(0, CHUNK)],
            out_p.at[:, pl.ds(tcg * 8, 8), :],
            ssems[p % NBUF])

    for step in range(NTASK):
        if step >= NBUF:
            scopies[step - NBUF].wait()
        start_gather(step)
        d = step - (NBUF - 1)
        if d >= 0:
            start_store(d)
    for d in range(NTASK - NBUF + 1, NTASK):
        start_store(d)
    for d in range(NTASK - NBUF, NTASK):
        scopies[d].wait()


_ENT_BLK = 8192
_ENT_GRID = -(-NUM_ENTITIES // _ENT_BLK)


def _ent_repack_body(tin, dep_a, dep_b, tout):
    del dep_a, dep_b
    x = tin[...]
    p = (lax.broadcasted_iota(jnp.int32, (EMBED_DIM, PADDED), 0)
         == lax.broadcasted_iota(jnp.int32, (EMBED_DIM, PADDED), 1)
         ).astype(jnp.float32)
    tout[...] = lax.dot_general(
        x, p, (((0,), (0,)), ((), ())),
        precision=lax.Precision.HIGHEST,
        preferred_element_type=jnp.float32)


_ent_repack = pl.pallas_call(
    _ent_repack_body,
    grid=(_ENT_GRID,),
    in_specs=[pl.BlockSpec((EMBED_DIM, _ENT_BLK), lambda i: (0, i)),
              pl.BlockSpec((NUM_RELATIONS, PADDED), lambda i: (0, 0)),
              pl.BlockSpec((NUM_TIMESTAMPS, PADDED), lambda i: (0, 0))],
    out_specs=pl.BlockSpec((_ENT_BLK, PADDED), lambda i: (i, 0)),
    out_shape=jax.ShapeDtypeStruct((NUM_ENTITIES, PADDED), jnp.float32),
)

def _untile(o):
    o4 = o.reshape(8, NTC, 8, CHUNK)
    return o4.transpose(1, 3, 0, 2).reshape(BATCH, EMBED_DIM)


def kernel(head, relation, tail, timestamp,
           entity_table, relation_table, timestamp_table):
    shp = (NTC, CHUNK)
    h2 = head.astype(jnp.int32).reshape(shp)
    t2 = tail.astype(jnp.int32).reshape(shp)
    r2 = relation.astype(jnp.int32).reshape(shp)
    s2 = timestamp.astype(jnp.int32).reshape(shp)

    pad = ((0, 0), (0, PADDED - EMBED_DIM))
    rel_p = jnp.pad(relation_table, pad)
    ts_p = jnp.pad(timestamp_table, pad)
    ent_lin = _ent_repack(entity_table.T, rel_p, ts_p)
    o_rel, o_ts = _pair_gather(r2, s2, rel_p, ts_p)
    h2b, t2b, _ = lax.optimization_barrier((h2, t2, o_rel))
    o_head, o_tail = _pair_gather(h2b, t2b, ent_lin, ent_lin)

    return (_untile(o_head), _untile(o_rel), _untile(o_tail), _untile(o_ts))


# speedup vs baseline: 1.1685x; 1.1677x over previous
import functools

import jax
import jax.numpy as jnp
from jax import lax
from jax.experimental import pallas as pl
from jax.experimental.pallas import tpu as pltpu
from jax.experimental.pallas import tpu_sc as plsc

NUM_ENTITIES = 100000
NUM_RELATIONS = 64
NUM_TIMESTAMPS = 1024
EMBED_DIM = 64
BATCH = 16384

NC = 2
NS = 16
NW = NC * NS
B_PER_W = BATCH // NW
CHUNK = 128
NCHUNK = B_PER_W // CHUNK
NOPS = 2
NTASK = NOPS * NCHUNK
NBUF = 4
PADDED = 128
NTC = BATCH // CHUNK

_MESH = plsc.VectorSubcoreMesh(core_axis_name="c", subcore_axis_name="s")

_OUT_SHAPE = jax.ShapeDtypeStruct((8, NTC * 8, CHUNK), jnp.float32)

_SCRATCH = (
    [pltpu.VMEM((NTASK, CHUNK), jnp.int32)]
    + [pltpu.VMEM((CHUNK, PADDED), jnp.float32) for _ in range(NBUF)]
    + [pltpu.VMEM((8, 8, 129), jnp.float32) for _ in range(NBUF)]
    + [pltpu.SemaphoreType.DMA for _ in range(2 * NBUF + NOPS)]
)


@functools.partial(
    pl.kernel,
    out_type=(_OUT_SHAPE, _OUT_SHAPE),
    mesh=_MESH,
    scratch_types=_SCRATCH,
    compiler_params=pltpu.CompilerParams(
        use_tc_tiling_on_sc=False, needs_layout_passes=False),
)
def _pair_gather(h2, t2, tbl_a, tbl_b,
                 out_h, out_t, idx_v,
                   b0, b1, b2, b3,
                   t0, t1, t2b, t3,
                   g0, g1, g2, g3,
                   s0, s1, s2s, s3,
                   i0, i1):
    wid = lax.axis_index("s") * NC + lax.axis_index("c")
    rowbase = wid * NCHUNK

    bufs = [b0, b1, b2, b3]
    tbufs = [t0, t1, t2b, t3]
    gsems = [g0, g1, g2, g3]
    ssems = [s0, s1, s2s, s3]
    isems = [i0, i1]
    idx_srcs = [h2, t2]
    tbls = [tbl_a, tbl_b]
    outs = [out_h, out_t]

    icopies = []
    for op in range(NOPS):
        icopies.append(pltpu.async_copy(
            idx_srcs[op].at[pl.ds(rowbase, NCHUNK)],
            idx_v.at[pl.ds(op * NCHUNK, NCHUNK)], isems[op]))
    idx_ready = [False] * NOPS

    tasks = [
        (outs[op], op, op * NCHUNK + c, c)
        for op in range(NOPS)
        for c in range(NCHUNK)
    ]

    gcopies = [None] * NTASK
    scopies = [None] * NTASK

    def start_gather(p):
        _, op, irow, _ = tasks[p]
        if not idx_ready[op]:
            icopies[op].wait()
            idx_ready[op] = True
        gcopies[p] = pltpu.async_copy(
            tbls[op].at[idx_v.at[irow]], bufs[p % NBUF], gsems[p % NBUF])

    iota16 = lax.iota(jnp.int32, 16)
    dgroups = []
    for d0 in range(0, EMBED_DIM, 16):
        dvec = d0 + iota16
        dgroups.append((dvec, dvec // 8, dvec % 8))

    def transpose_chunk(buf, tbuf):
        def body(c, carry):
            cc = jnp.full((16,), c, jnp.int32)
            for dvec, trh, srh in dgroups:
                v = plsc.load_gather(buf, [cc, dvec])
                plsc.store_scatter(tbuf, [trh, srh, cc], v)
            return carry
        lax.fori_loop(0, CHUNK, body, jnp.int32(0), unroll=4)

    def start_store(p):
        out_p, _, _, c = tasks[p]
        gcopies[p].wait()
        transpose_chunk(bufs[p % NBUF], tbufs[p % NBUF])
        tcg = wid * NCHUNK + c
        scopies[p] = pltpu.async_copy(
            tbufs[p % NBUF].at[:, :, pl.ds(0, CHUNK)],
            out_p.at[:, pl.ds(tcg * 8, 8), :],
            ssems[p % NBUF])

    for step in range(NTASK):
        if step >= NBUF:
            scopies[step - NBUF].wait()
        start_gather(step)
        d = step - (NBUF - 1)
        if d >= 0:
            start_store(d)
    for d in range(NTASK - NBUF + 1, NTASK):
        start_store(d)
    for d in range(NTASK - NBUF, NTASK):
        scopies[d].wait()


_ENT_BLK = 8192
_ENT_GRID = -(-NUM_ENTITIES // _ENT_BLK)


def _ent_repack_body(tin, tout):
    x = tin[...]
    p = (lax.broadcasted_iota(jnp.int32, (EMBED_DIM, PADDED), 0)
         == lax.broadcasted_iota(jnp.int32, (EMBED_DIM, PADDED), 1)
         ).astype(jnp.float32)
    tout[...] = lax.dot_general(
        x, p, (((0,), (0,)), ((), ())),
        precision=lax.Precision.HIGHEST,
        preferred_element_type=jnp.float32)


_ent_repack = pl.pallas_call(
    _ent_repack_body,
    grid=(_ENT_GRID,),
    in_specs=[pl.BlockSpec((EMBED_DIM, _ENT_BLK), lambda i: (0, i))],
    out_specs=pl.BlockSpec((_ENT_BLK, PADDED), lambda i: (i, 0)),
    out_shape=jax.ShapeDtypeStruct((NUM_ENTITIES, PADDED), jnp.float32),
)


_TCSUB = 8


def _small_lookup_body(ridx, tidx, rel_t, ts_t, orel, ots):
    for idxref, tblref, outref, nv in (
            (ridx, rel_t, orel, NUM_RELATIONS),
            (tidx, ts_t, ots, NUM_TIMESTAMPS)):
        pieces = []
        for tcl in range(_TCSUB):
            idx = idxref[tcl].reshape(1, CHUNK)
            pieces.append(
                (lax.broadcasted_iota(jnp.int32, (nv, CHUNK), 0)
                 == idx).astype(jnp.float32))
        onehot = jnp.concatenate(pieces, axis=1)
        ot = jnp.dot(tblref[...], onehot,
                     precision=lax.Precision.HIGHEST,
                     preferred_element_type=jnp.float32)
        for tcl in range(_TCSUB):
            outref[:, tcl * 8:(tcl + 1) * 8, :] = (
                ot[:, tcl * CHUNK:(tcl + 1) * CHUNK].reshape(8, 8, CHUNK))


_small_lookup = pl.pallas_call(
    _small_lookup_body,
    grid=(NTC // _TCSUB,),
    in_specs=[
        pl.BlockSpec((_TCSUB, 1, CHUNK), lambda i: (i, 0, 0)),
        pl.BlockSpec((_TCSUB, 1, CHUNK), lambda i: (i, 0, 0)),
        pl.BlockSpec((EMBED_DIM, NUM_RELATIONS), lambda i: (0, 0)),
        pl.BlockSpec((EMBED_DIM, NUM_TIMESTAMPS), lambda i: (0, 0)),
    ],
    out_specs=[
        pl.BlockSpec((8, _TCSUB * 8, CHUNK), lambda i: (0, i, 0)),
        pl.BlockSpec((8, _TCSUB * 8, CHUNK), lambda i: (0, i, 0)),
    ],
    out_shape=[_OUT_SHAPE, _OUT_SHAPE],
)


def _untile(o):
    o4 = o.reshape(8, NTC, 8, CHUNK)
    return o4.transpose(1, 3, 0, 2).reshape(BATCH, EMBED_DIM)


def kernel(head, relation, tail, timestamp,
           entity_table, relation_table, timestamp_table):
    shp = (NTC, CHUNK)
    h2 = head.astype(jnp.int32).reshape(shp)
    t2 = tail.astype(jnp.int32).reshape(shp)
    r3 = relation.astype(jnp.int32).reshape(NTC, 1, CHUNK)
    s3 = timestamp.astype(jnp.int32).reshape(NTC, 1, CHUNK)

    ent_lin = _ent_repack(entity_table.T)
    o_rel, o_ts = _small_lookup(r3, s3, relation_table.T, timestamp_table.T)
    o_head, o_tail = _pair_gather(h2, t2, ent_lin, ent_lin)

    return (_untile(o_head), _untile(o_rel), _untile(o_tail), _untile(o_ts))


# speedup vs baseline: 1.2938x; 1.1073x over previous
import functools

import jax
import jax.numpy as jnp
from jax import lax
from jax.experimental import pallas as pl
from jax.experimental.pallas import tpu as pltpu
from jax.experimental.pallas import tpu_sc as plsc

NUM_ENTITIES = 100000
NUM_RELATIONS = 64
NUM_TIMESTAMPS = 1024
EMBED_DIM = 64
BATCH = 16384

NC = 2
NS = 16
NW = NC * NS
B_PER_W = BATCH // NW
CHUNK = 128
NCHUNK = B_PER_W // CHUNK
NBUF = 4
PADDED = 128
NTC = BATCH // CHUNK

_MESH = plsc.VectorSubcoreMesh(core_axis_name="c", subcore_axis_name="s")

_OUT_SHAPE = jax.ShapeDtypeStruct((8, NTC * 8, CHUNK), jnp.float32)

def _make_gather(nops):
    ntask = nops * NCHUNK
    scratch = (
        [pltpu.VMEM((ntask, CHUNK), jnp.int32)]
        + [pltpu.VMEM((CHUNK, PADDED), jnp.float32) for _ in range(NBUF)]
        + [pltpu.VMEM((8, 8, 129), jnp.float32) for _ in range(NBUF)]
        + [pltpu.SemaphoreType.DMA for _ in range(2 * NBUF + nops)]
    )

    @functools.partial(
        pl.kernel,
        out_type=tuple(_OUT_SHAPE for _ in range(nops)),
        mesh=_MESH,
        scratch_types=scratch,
        compiler_params=pltpu.CompilerParams(
            use_tc_tiling_on_sc=False, needs_layout_passes=False),
    )
    def _gather(*refs):
        idx_srcs = refs[:nops]
        tbls = refs[nops:2 * nops]
        outs = refs[2 * nops:3 * nops]
        idx_v = refs[3 * nops]
        bufs = list(refs[3 * nops + 1:3 * nops + 1 + NBUF])
        tbufs = list(refs[3 * nops + 1 + NBUF:3 * nops + 1 + 2 * NBUF])
        gsems = list(refs[3 * nops + 1 + 2 * NBUF:3 * nops + 1 + 3 * NBUF])
        ssems = list(refs[3 * nops + 1 + 3 * NBUF:3 * nops + 1 + 4 * NBUF])
        isems = list(refs[3 * nops + 1 + 4 * NBUF:])

        wid = lax.axis_index("s") * NC + lax.axis_index("c")
        rowbase = wid * NCHUNK

        icopies = []
        for op in range(nops):
            icopies.append(pltpu.async_copy(
                idx_srcs[op].at[pl.ds(rowbase, NCHUNK)],
                idx_v.at[pl.ds(op * NCHUNK, NCHUNK)], isems[op]))
        idx_ready = [False] * nops

        tasks = [
            (outs[op], op, op * NCHUNK + c, c)
            for op in range(nops)
            for c in range(NCHUNK)
        ]

        gcopies = [None] * ntask
        scopies = [None] * ntask

        def start_gather(p):
            _, op, irow, _ = tasks[p]
            if not idx_ready[op]:
                icopies[op].wait()
                idx_ready[op] = True
            gcopies[p] = pltpu.async_copy(
                tbls[op].at[idx_v.at[irow]], bufs[p % NBUF], gsems[p % NBUF])

        iota16 = lax.iota(jnp.int32, 16)
        dgroups = []
        for d0 in range(0, EMBED_DIM, 16):
            dvec = d0 + iota16
            dgroups.append((dvec, dvec // 8, dvec % 8))

        def transpose_chunk(buf, tbuf):
            def body(c, carry):
                cc = jnp.full((16,), c, jnp.int32)
                for dvec, trh, srh in dgroups:
                    v = plsc.load_gather(buf, [cc, dvec])
                    plsc.store_scatter(tbuf, [trh, srh, cc], v)
                return carry
            lax.fori_loop(0, CHUNK, body, jnp.int32(0), unroll=4)

        def start_store(p):
            out_p, _, _, c = tasks[p]
            gcopies[p].wait()
            transpose_chunk(bufs[p % NBUF], tbufs[p % NBUF])
            tcg = wid * NCHUNK + c
            scopies[p] = pltpu.async_copy(
                tbufs[p % NBUF].at[:, :, pl.ds(0, CHUNK)],
                out_p.at[:, pl.ds(tcg * 8, 8), :],
                ssems[p % NBUF])

        for step in range(ntask):
            if step >= NBUF:
                scopies[step - NBUF].wait()
            start_gather(step)
            d = step - (NBUF - 1)
            if d >= 0:
                start_store(d)
        for d in range(ntask - NBUF + 1, ntask):
            start_store(d)
        for d in range(ntask - NBUF, ntask):
            scopies[d].wait()

    return _gather


_gather3 = _make_gather(3)


_ENT_BLK = 8192
_ENT_GRID = -(-NUM_ENTITIES // _ENT_BLK)


def _ent_repack_body(tin, tout):
    x = tin[...]
    p = (lax.broadcasted_iota(jnp.int32, (EMBED_DIM, PADDED), 0)
         == lax.broadcasted_iota(jnp.int32, (EMBED_DIM, PADDED), 1)
         ).astype(jnp.float32)
    tout[...] = lax.dot_general(
        x, p, (((0,), (0,)), ((), ())),
        precision=lax.Precision.HIGHEST,
        preferred_element_type=jnp.float32)


_ent_repack = pl.pallas_call(
    _ent_repack_body,
    grid=(_ENT_GRID,),
    in_specs=[pl.BlockSpec((EMBED_DIM, _ENT_BLK), lambda i: (0, i))],
    out_specs=pl.BlockSpec((_ENT_BLK, PADDED), lambda i: (i, 0)),
    out_shape=jax.ShapeDtypeStruct((NUM_ENTITIES, PADDED), jnp.float32),
)


_TCSUB = 8


def _rel_lookup_body(ridx, rel_t, orel):
    pieces = []
    for tcl in range(_TCSUB):
        idx = ridx[tcl].reshape(1, CHUNK)
        pieces.append(
            (lax.broadcasted_iota(jnp.int32, (NUM_RELATIONS, CHUNK), 0)
             == idx).astype(jnp.float32))
    onehot = jnp.concatenate(pieces, axis=1)
    ot = jnp.dot(rel_t[...], onehot,
                 precision=lax.Precision.HIGHEST,
                 preferred_element_type=jnp.float32)
    for tcl in range(_TCSUB):
        orel[:, tcl * 8:(tcl + 1) * 8, :] = (
            ot[:, tcl * CHUNK:(tcl + 1) * CHUNK].reshape(8, 8, CHUNK))


_rel_lookup = pl.pallas_call(
    _rel_lookup_body,
    grid=(NTC // _TCSUB,),
    in_specs=[
        pl.BlockSpec((_TCSUB, 1, CHUNK), lambda i: (i, 0, 0)),
        pl.BlockSpec((EMBED_DIM, NUM_RELATIONS), lambda i: (0, 0)),
    ],
    out_specs=pl.BlockSpec((8, _TCSUB * 8, CHUNK), lambda i: (0, i, 0)),
    out_shape=_OUT_SHAPE,
)


def _untile(o):
    o4 = o.reshape(8, NTC, 8, CHUNK)
    return o4.transpose(1, 3, 0, 2).reshape(BATCH, EMBED_DIM)


def kernel(head, relation, tail, timestamp,
           entity_table, relation_table, timestamp_table):
    shp = (NTC, CHUNK)
    h2 = head.astype(jnp.int32).reshape(shp)
    t2 = tail.astype(jnp.int32).reshape(shp)
    s2 = timestamp.astype(jnp.int32).reshape(shp)
    r3 = relation.astype(jnp.int32).reshape(NTC, 1, CHUNK)

    ent_lin = _ent_repack(entity_table.T)
    ts_p = jnp.pad(timestamp_table, ((0, 0), (0, PADDED - EMBED_DIM)))
    o_rel = _rel_lookup(r3, relation_table.T)
    o_head, o_tail, o_ts = _gather3(h2, t2, s2, ent_lin, ent_lin, ts_p)

    return (_untile(o_head), _untile(o_rel), _untile(o_tail), _untile(o_ts))
